# baseline (speedup 1.0000x reference)
import functools

import jax
import jax.numpy as jnp
from jax.experimental import pallas as pl
from jax.experimental.pallas import tpu as pltpu

_N = 2048
_D = 128
_BM = 512


def _qt_x_kernel(q_ref, x_ref, o_ref):
    o_ref[...] = jax.lax.dot_general(
        q_ref[...], x_ref[...], (((0,), (0,)), ((), ())),
        preferred_element_type=jnp.float32)


def _q_fz_kernel(q_ref, f_ref, y_ref, o_ref):
    z = f_ref[...] * y_ref[...]
    o_ref[...] = jnp.abs(jax.lax.dot_general(
        q_ref[...], z, (((1,), (0,)), ((), ())),
        preferred_element_type=jnp.float32))


def _qt_matmul(q, x):
    n, d = x.shape
    return pl.pallas_call(
        _qt_x_kernel,
        grid=(n // _BM,),
        in_specs=[
            pl.BlockSpec((n, _BM), lambda i: (0, i)),
            pl.BlockSpec((n, d), lambda i: (0, 0)),
        ],
        out_specs=pl.BlockSpec((_BM, d), lambda i: (i, 0)),
        out_shape=jax.ShapeDtypeStruct((n, d), jnp.float32),
    )(q, x)


def _wavelet_apply(q, fbig, y, n_f, n_yblk):
    n = q.shape[0]
    nj = n_f * n_yblk
    return pl.pallas_call(
        _q_fz_kernel,
        grid=(n // _BM, nj),
        in_specs=[
            pl.BlockSpec((_BM, n), lambda i, j: (i, 0)),
            pl.BlockSpec((n, _D), lambda i, j: (0, j // n_yblk)),
            pl.BlockSpec((n, _D), lambda i, j: (0, j % n_yblk)),
        ],
        out_specs=pl.BlockSpec((_BM, _D), lambda i, j: (i, j)),
        out_shape=jax.ShapeDtypeStruct((n, nj * _D), jnp.float32),
    )(q, fbig, y)


def kernel(x, edge_index):
    n = x.shape[0]
    d = x.shape[1]
    row = edge_index[0]
    col = edge_index[1]
    edge_weight = jnp.ones((edge_index.shape[1],), dtype=x.dtype)
    deg = jnp.zeros((n,), dtype=x.dtype).at[col].add(edge_weight)
    deg_half = deg ** (-0.5)
    deg_half = jnp.where(jnp.isinf(deg_half), 0.0, deg_half)
    w = deg_half[row] * deg_half[col]
    t = jnp.zeros((n, n), dtype=x.dtype).at[row, col].add(w)

    evals, q = jnp.linalg.eigh(t)
    l2 = evals * evals
    l4 = l2 * l2
    l8 = l4 * l4
    l16 = l8 * l8
    fmat = jnp.stack([
        evals - l2, l2 - l4, l4 - l8, l8 - l16
    ], axis=1)
    fmat = jnp.sqrt(jnp.clip(fmat, 0.0, None))
    fbig = jnp.repeat(fmat, _D, axis=1)

    y1 = _qt_matmul(q, x)
    s1_mine = _wavelet_apply(q, fbig, y1, 4, 1)
    y2 = _qt_matmul(q, s1_mine)
    s2_mine = _wavelet_apply(q, fbig, y2, 4, 4)

    s1_ref = s1_mine.reshape(n, 4, d).transpose(0, 2, 1).reshape(n, 4 * d)
    s2_ref = s2_mine.reshape(n, 4, 4, d).transpose(0, 3, 2, 1).reshape(n, 16 * d)
    return jnp.concatenate([x, s1_ref, s2_ref], axis=1)


# speedup vs baseline: 5.9655x; 5.9655x over previous
import functools

import jax
import jax.numpy as jnp
from jax import lax
from jax.experimental import pallas as pl
from jax.experimental.pallas import tpu as pltpu

_N = 2048
_D = 128
_K = 384
_PIT = 24


def _power_kernel(t_ref, v_ref, rho_ref):
    def body(_, v):
        w = jnp.dot(t_ref[...], v, preferred_element_type=jnp.float32)
        nrm = jnp.sqrt(jnp.sum(w * w, axis=0, keepdims=True))
        return w / jnp.maximum(nrm, 1e-30)
    v = body(0, v_ref[...])
    v = lax.fori_loop(0, _PIT - 1, body, v)
    w = jnp.dot(t_ref[...], v, preferred_element_type=jnp.float32)
    nrm = jnp.sqrt(jnp.sum(w * w, axis=0))
    rho_ref[0, 0] = jnp.max(nrm)


def _estimate_rho(ts):
    n = ts.shape[0]
    i = jnp.arange(n, dtype=jnp.float32)
    cols = [jnp.ones((n,), jnp.float32)]
    for p in (1.0, 2.0, 3.0, 5.0, 7.0, 11.0, 13.0):
        cols.append(jnp.sin(0.7318 * p * i + 0.25 * p))
    v0 = jnp.stack(cols, axis=1)
    v0 = v0 / jnp.sqrt(jnp.sum(v0 * v0, axis=0, keepdims=True))
    rho = pl.pallas_call(
        _power_kernel,
        out_shape=jax.ShapeDtypeStruct((1, 1), jnp.float32),
        in_specs=[
            pl.BlockSpec(memory_space=pltpu.VMEM),
            pl.BlockSpec(memory_space=pltpu.VMEM),
        ],
        out_specs=pl.BlockSpec(memory_space=pltpu.SMEM),
    )(ts, v0)
    return rho[0, 0]


def _cheb_stage_kernel(t_ref, v_ref, c_ref, o_ref, t0, t1):
    t0[...] = v_ref[...]
    t1[...] = jnp.dot(t_ref[...], v_ref[...],
                      preferred_element_type=jnp.float32)
    for i in range(4):
        o_ref[i] = c_ref[0, i] * t0[...] + c_ref[1, i] * t1[...]

    def body(j, _):
        a = jnp.dot(t_ref[...], t1[...], preferred_element_type=jnp.float32)
        t0[...] = 2.0 * a - t0[...]
        for i in range(4):
            o_ref[i] += c_ref[2 * j, i] * t0[...]
        b = jnp.dot(t_ref[...], t0[...], preferred_element_type=jnp.float32)
        t1[...] = 2.0 * b - t1[...]
        for i in range(4):
            o_ref[i] += c_ref[2 * j + 1, i] * t1[...]
        return 0

    lax.fori_loop(1, (_K + 2) // 2, body, 0)
    for i in range(4):
        o_ref[i] = jnp.abs(o_ref[i])


def _cheb_apply(ts_scaled, v, coefs):
    n, w = v.shape
    nblk = w // _D
    grid = (nblk,)
    return pl.pallas_call(
        _cheb_stage_kernel,
        grid=grid,
        in_specs=[
            pl.BlockSpec((n, n), lambda j: (0, 0)),
            pl.BlockSpec((n, _D), lambda j: (0, j)),
            pl.BlockSpec(memory_space=pltpu.SMEM),
        ],
        out_specs=pl.BlockSpec((4, n, _D), lambda j: (0, 0, j)),
        out_shape=jax.ShapeDtypeStruct((4, n, w), jnp.float32),
        scratch_shapes=[
            pltpu.VMEM((n, _D), jnp.float32),
            pltpu.VMEM((n, _D), jnp.float32),
        ],
    )(ts_scaled, v, coefs)


def kernel(x, edge_index):
    n = x.shape[0]
    d = x.shape[1]
    row = edge_index[0]
    col = edge_index[1]
    ones = jnp.ones((edge_index.shape[1],), dtype=x.dtype)
    deg = jnp.zeros((n,), dtype=x.dtype).at[col].add(ones)
    deg_half = deg ** (-0.5)
    deg_half = jnp.where(jnp.isinf(deg_half), 0.0, deg_half)
    w = deg_half[row] * deg_half[col]
    t = jnp.zeros((n, n), dtype=x.dtype).at[row, col].add(w)
    ts = 0.5 * (t + t.T)

    gersh = jnp.max(jnp.sum(jnp.abs(ts), axis=1))
    rho = _estimate_rho(ts)
    dom = jnp.minimum(gersh, jnp.maximum(rho * 1.06, 1.12))

    k = jnp.arange(_K + 1, dtype=jnp.float32)
    xs = jnp.cos(jnp.pi * (k + 0.5) / (_K + 1))
    ls = dom * xs
    l2 = ls * ls
    l4 = l2 * l2
    l8 = l4 * l4
    l16 = l8 * l8
    gvals = jnp.stack([
        jnp.sqrt(jnp.clip(ls - l2, 0.0, None)),
        jnp.sqrt(jnp.clip(l2 - l4, 0.0, None)),
        jnp.sqrt(jnp.clip(l4 - l8, 0.0, None)),
        jnp.sqrt(jnp.clip(l8 - l16, 0.0, None)),
    ], axis=0)
    j = jnp.arange(_K + 1, dtype=jnp.float32)
    cosm = jnp.cos(jnp.pi * j[:, None] * (k[None, :] + 0.5) / (_K + 1))
    coefs = (2.0 / (_K + 1)) * (gvals @ cosm.T)
    coefs = coefs.at[:, 0].mul(0.5)
    coefs = jnp.pad(coefs, ((0, 0), (0, 1)))
    coefs_t = coefs.T

    ts_scaled = ts / dom

    s1_3 = _cheb_apply(ts_scaled, x, coefs_t)
    s1_mine = s1_3.transpose(1, 0, 2).reshape(n, 4 * d)
    s2_3 = _cheb_apply(ts_scaled, s1_mine, coefs_t)

    s1_ref = s1_3.transpose(1, 2, 0).reshape(n, 4 * d)
    s2_ref = (s2_3.reshape(4, n, 4, d)
              .transpose(1, 3, 2, 0).reshape(n, 16 * d))
    return jnp.concatenate([x, s1_ref, s2_ref], axis=1)
